# Initial kernel scaffold; baseline (speedup 1.0000x reference)
import jax
import jax.numpy as jnp
from jax.experimental import pallas as pl


def kernel(x, edge_index, params):
    raise NotImplementedError("write your pallas kernel here")


# speedup vs baseline: 22.8013x; 22.8013x over previous
import functools

import jax
import jax.numpy as jnp
from jax import lax
from jax.experimental import pallas as pl
from jax.experimental.pallas import tpu as pltpu
from jax.experimental.pallas import tpu_sc as plsc

N = 10000
E = 320000
K = 80
E2 = E // K
NC = 2
NS = 16
NW = NC * NS
NP1 = 10240
LANES = 16

D1, D2, D3, DMU = 94, 61, 38, 16
D1P, D2P, D3P = 96, 64, 48

f32 = jnp.float32
i32 = jnp.int32


def _tc(body, out_shape, *args):
    return pl.pallas_call(body, out_shape=out_shape)(*args)


def _gat_head(h, avs, avd):
    asv = jnp.sum(h * avs[None, :], axis=1)
    adv = jnp.sum(h * avd[None, :], axis=1)
    a = asv + adv
    a = jnp.where(a >= 0, a, 0.2 * a)
    return asv, adv, jnp.exp(a)


def tc_gat_pre(x, w, avs, avd):
    chp = w.shape[1]

    def body(x_r, w_r, avs_r, avd_r, h_r, as_r, ad_r, ps_r):
        h = jnp.dot(x_r[...], w_r[...], preferred_element_type=f32)
        h_r[...] = h
        asv, adv, ps = _gat_head(h, avs_r[...], avd_r[...])
        as_r[...] = asv
        ad_r[...] = adv
        ps_r[...] = ps

    return _tc(body, (
        jax.ShapeDtypeStruct((N, chp), f32),
        jax.ShapeDtypeStruct((N,), f32),
        jax.ShapeDtypeStruct((N,), f32),
        jax.ShapeDtypeStruct((N,), f32),
    ), x, w, avs, avd)


def _gat_prev(acc_r, h_r, ps_r, inv_r, b_r):
    wself = ps_r[...] * inv_r[...]
    return (acc_r[0, :N] + acc_r[1, :N]
            + wself[:, None] * h_r[...]) + b_r[...][None, :]


def tc_inv(den, ps):
    def body(den_r, ps_r, inv_r):
        den_t = den_r[0, :N] + den_r[1, :N] + ps_r[...]
        inv_r[...] = 1.0 / jnp.maximum(den_t, 1e-16)

    return _tc(body, jax.ShapeDtypeStruct((N,), f32), den, ps)


def tc_gat_combine_next(acc, h, ps, inv, b, wn, avs, avd):
    chn = wn.shape[1]

    def body(acc_r, h_r, ps_r, inv_r, b_r, wn_r, avs_r, avd_r,
             h2_r, as_r, ad_r, ps2_r):
        prev = _gat_prev(acc_r, h_r, ps_r, inv_r, b_r)
        e = jnp.maximum(prev, 0.0)
        h2 = jnp.dot(e, wn_r[...], preferred_element_type=f32)
        h2_r[...] = h2
        asv, adv, ps2 = _gat_head(h2, avs_r[...], avd_r[...])
        as_r[...] = asv
        ad_r[...] = adv
        ps2_r[...] = ps2

    return _tc(body, (
        jax.ShapeDtypeStruct((N, chn), f32),
        jax.ShapeDtypeStruct((N,), f32),
        jax.ShapeDtypeStruct((N,), f32),
        jax.ShapeDtypeStruct((N,), f32),
    ), acc, h, ps, inv, b, wn, avs, avd)


def tc_gat_final(acc, h, ps, inv, b):
    def body(acc_r, h_r, ps_r, inv_r, b_r, out_r):
        prev = _gat_prev(acc_r, h_r, ps_r, inv_r, b_r)
        out_r[...] = jax.nn.sigmoid(prev)

    return _tc(body, jax.ShapeDtypeStruct((N, DMU), f32), acc, h, ps, inv, b)


def tc_gcn_pre(x, w, deg):
    dout = w.shape[1]
    hf = dout // 2

    def body(x_r, w_r, deg_r, dinv_r, ga_r, gb_r):
        deg_t = deg_r[0, :N] + deg_r[1, :N] + 1.0
        dinv = lax.rsqrt(deg_t)
        dinv_r[...] = dinv
        g = dinv[:, None] * jnp.dot(x_r[...], w_r[...],
                                    preferred_element_type=f32)
        ga_r[...] = g[:, :hf]
        gb_r[...] = g[:, hf:]

    return _tc(body, (
        jax.ShapeDtypeStruct((N,), f32),
        jax.ShapeDtypeStruct((N, hf), f32),
        jax.ShapeDtypeStruct((N, hf), f32),
    ), x, w, deg)


def tc_gcn_combine_next(o, ga, gb, dinv, b, wn):
    dout = wn.shape[1]
    hf = dout // 2

    def body(o_r, ga_r, gb_r, dinv_r, b_r, wn_r, ga2_r, gb2_r):
        sc = jnp.concatenate([o_r[0, :N], o_r[1, :N]], axis=1)
        g = jnp.concatenate([ga_r[...], gb_r[...]], axis=1)
        dinv = dinv_r[...]
        prev = dinv[:, None] * (sc + g) + b_r[...][None, :]
        e = jnp.maximum(prev, 0.0)
        g2 = dinv[:, None] * jnp.dot(e, wn_r[...],
                                     preferred_element_type=f32)
        ga2_r[...] = g2[:, :hf]
        gb2_r[...] = g2[:, hf:]

    return _tc(body, (
        jax.ShapeDtypeStruct((N, hf), f32),
        jax.ShapeDtypeStruct((N, hf), f32),
    ), o, ga, gb, dinv, b, wn)


def tc_gcn_final(o, ga, gb, dinv, b):
    def body(o_r, ga_r, gb_r, dinv_r, b_r, out_r):
        sc = jnp.concatenate([o_r[0, :N], o_r[1, :N]], axis=1)
        g = jnp.concatenate([ga_r[...], gb_r[...]], axis=1)
        prev = dinv_r[...][:, None] * (sc + g) + b_r[...][None, :]
        out_r[...] = jax.nn.sigmoid(prev)

    return _tc(body, jax.ShapeDtypeStruct((N, 2 * ga.shape[1]), f32),
               o, ga, gb, dinv, b)


_MESH = plsc.VectorSubcoreMesh(core_axis_name="c", subcore_axis_name="s")
_SC_PARAMS = pltpu.CompilerParams(use_tc_tiling_on_sc=False,
                                  needs_layout_passes=False)

_RPW_GAT = E2 // NW
_RPW_GCN = E2 // NS
_STRIPE1 = NP1 // NS
_STRIPEN = N // NS


def _zero_1d(zb, acc, s):
    @pl.loop(0, _STRIPE1 // LANES)
    def _(t):
        zb[pl.ds(pl.multiple_of(t * LANES, LANES), LANES)] = jnp.zeros(
            (LANES,), f32)
    pltpu.sync_copy(zb, acc.at[pl.ds(s * _STRIPE1, _STRIPE1)])


def _zero_rows(buf, acc, s, chp):
    @pl.loop(0, K * (chp // LANES))
    def _(t):
        r = t // (chp // LANES)
        v = t % (chp // LANES)
        buf[r, pl.ds(pl.multiple_of(v * LANES, LANES), LANES)] = jnp.zeros(
            (LANES,), f32)

    @pl.loop(0, _STRIPE1 // K)
    def _(t):
        pltpu.sync_copy(buf, acc.at[pl.ds(s * _STRIPE1 + t * K, K)])


def _sc_alpha_body(with_deg, src_h, dst_h, as_h, ad_h, p_h, den_h, deg_h,
                   srcv, dstv, at, dt, pbuf, zb, ones, den_acc, deg_acc):
    c = lax.axis_index("c")
    s = lax.axis_index("s")
    wid = c * NS + s
    row0 = wid * _RPW_GAT
    pltpu.sync_copy(src_h.at[pl.ds(row0, _RPW_GAT)], srcv)
    pltpu.sync_copy(dst_h.at[pl.ds(row0, _RPW_GAT)], dstv)
    pltpu.sync_copy(as_h, at)
    pltpu.sync_copy(ad_h, dt)

    _zero_1d(zb, den_acc, s)
    if with_deg:
        pltpu.sync_copy(zb, deg_acc.at[pl.ds(s * _STRIPE1, _STRIPE1)])
        for g in range(K // LANES):
            ones[pl.ds(g * LANES, LANES)] = jnp.ones((LANES,), f32)
    plsc.subcore_barrier()

    @pl.loop(0, _RPW_GAT)
    def _(i):
        for g in range(K // LANES):
            off = g * LANES
            s16 = srcv[i, pl.ds(off, LANES)]
            d16 = dstv[i, pl.ds(off, LANES)]
            a = plsc.load_gather(at, [s16]) + plsc.load_gather(dt, [d16])
            a = jnp.where(a >= 0, a, 0.2 * a)
            pbuf[i, pl.ds(off, LANES)] = jnp.exp(a)
        pltpu.sync_copy(pbuf.at[i], den_acc.at[dstv.at[i]], add=True)
        if with_deg:
            pltpu.sync_copy(ones, deg_acc.at[dstv.at[i]], add=True)

    plsc.subcore_barrier()
    pltpu.sync_copy(pbuf, p_h.at[pl.ds(row0, _RPW_GAT)])
    pltpu.sync_copy(den_acc.at[pl.ds(s * _STRIPE1, _STRIPE1)],
                    den_h.at[c, pl.ds(s * _STRIPE1, _STRIPE1)])
    if with_deg:
        pltpu.sync_copy(deg_acc.at[pl.ds(s * _STRIPE1, _STRIPE1)],
                        deg_h.at[c, pl.ds(s * _STRIPE1, _STRIPE1)])


def _make_sc_alpha(with_deg):
    outs = [jax.ShapeDtypeStruct((E2, K), f32),
            jax.ShapeDtypeStruct((NC, NP1), f32)]
    if with_deg:
        outs.append(jax.ShapeDtypeStruct((NC, NP1), f32))
    scratch = [
        pltpu.VMEM((_RPW_GAT, K), i32),
        pltpu.VMEM((_RPW_GAT, K), i32),
        pltpu.VMEM((N,), f32),
        pltpu.VMEM((N,), f32),
        pltpu.VMEM((_RPW_GAT, K), f32),
        pltpu.VMEM((_STRIPE1,), f32),
        pltpu.VMEM((K,), f32),
        pltpu.VMEM_SHARED((NP1,), f32),
        pltpu.VMEM_SHARED((NP1,), f32),
    ]

    def body(src_h, dst_h, as_h, ad_h, *rest):
        if with_deg:
            p_h, den_h, deg_h = rest[:3]
            rest = rest[3:]
        else:
            p_h, den_h = rest[:2]
            deg_h = None
            rest = rest[2:]
        _sc_alpha_body(with_deg, src_h, dst_h, as_h, ad_h,
                       p_h, den_h, deg_h, *rest)

    return pl.kernel(body, out_type=tuple(outs), mesh=_MESH,
                     scratch_types=scratch, compiler_params=_SC_PARAMS)


_sc_alpha_deg = _make_sc_alpha(True)
_sc_alpha = _make_sc_alpha(False)


def _make_sc_rows_gat(chp):
    scratch = [
        pltpu.VMEM((_RPW_GAT, K), i32),
        pltpu.VMEM((_RPW_GAT, K), i32),
        pltpu.VMEM((_RPW_GAT, K), f32),
        pltpu.VMEM((N,), f32),
        pltpu.VMEM((K, chp), f32),
        pltpu.VMEM((K, chp), f32),
        pltpu.VMEM_SHARED((NP1, chp), f32),
    ]

    def body(h_h, src_h, dst_h, p_h, inv_h, acc_h,
             srcv, dstv, pv, inv, gbuf, sbuf, acc):
        c = lax.axis_index("c")
        s = lax.axis_index("s")
        wid = c * NS + s
        row0 = wid * _RPW_GAT
        pltpu.sync_copy(src_h.at[pl.ds(row0, _RPW_GAT)], srcv)
        pltpu.sync_copy(dst_h.at[pl.ds(row0, _RPW_GAT)], dstv)
        pltpu.sync_copy(p_h.at[pl.ds(row0, _RPW_GAT)], pv)
        pltpu.sync_copy(inv_h, inv)

        _zero_rows(sbuf, acc, s, chp)
        plsc.subcore_barrier()

        lane = lax.iota(i32, LANES)

        @pl.loop(0, _RPW_GAT)
        def _(i):
            pltpu.sync_copy(h_h.at[srcv.at[i]], gbuf)
            for g in range(K // LANES):
                off = g * LANES
                d16 = dstv[i, pl.ds(off, LANES)]
                w16 = pv[i, pl.ds(off, LANES)] * plsc.load_gather(inv, [d16])
                for j in range(LANES):
                    wj = jnp.sum(jnp.where(lane == j, w16, 0.0))
                    e = off + j
                    for v in range(chp // LANES):
                        sbuf[e, pl.ds(v * LANES, LANES)] = (
                            gbuf[e, pl.ds(v * LANES, LANES)] * wj)
            pltpu.sync_copy(sbuf, acc.at[dstv.at[i]], add=True)

        plsc.subcore_barrier()
        pltpu.sync_copy(acc.at[pl.ds(s * _STRIPE1, _STRIPE1)],
                        acc_h.at[c, pl.ds(s * _STRIPE1, _STRIPE1)])

    return pl.kernel(body,
                     out_type=jax.ShapeDtypeStruct((NC, NP1, chp), f32),
                     mesh=_MESH, scratch_types=scratch,
                     compiler_params=_SC_PARAMS)


_sc_rows_gat = {chp: _make_sc_rows_gat(chp) for chp in (D1P, D2P, D3P, DMU)}


def _make_sc_rows_gcn(hf):
    scratch = [
        pltpu.VMEM((_RPW_GAT, K), i32),
        pltpu.VMEM((_RPW_GAT, K), i32),
        pltpu.VMEM((K, hf), f32),
        pltpu.VMEM_SHARED((NP1, hf), f32),
    ]

    def body(g_h, src_h, dst_h, o_h, srcv, dstv, gbuf, acc):
        c = lax.axis_index("c")
        s = lax.axis_index("s")

        _zero_rows(gbuf, acc, s, hf)
        plsc.subcore_barrier()

        @pl.loop(0, _RPW_GCN // _RPW_GAT)
        def _(ph):
            row0 = s * _RPW_GCN + ph * _RPW_GAT
            pltpu.sync_copy(src_h.at[pl.ds(row0, _RPW_GAT)], srcv)
            pltpu.sync_copy(dst_h.at[pl.ds(row0, _RPW_GAT)], dstv)

            @pl.loop(0, _RPW_GAT)
            def _(i):
                pltpu.sync_copy(g_h.at[c].at[srcv.at[i]], gbuf)
                pltpu.sync_copy(gbuf, acc.at[dstv.at[i]], add=True)

        plsc.subcore_barrier()
        pltpu.sync_copy(acc.at[pl.ds(s * _STRIPE1, _STRIPE1)],
                        o_h.at[c, pl.ds(s * _STRIPE1, _STRIPE1)])

    return pl.kernel(body,
                     out_type=jax.ShapeDtypeStruct((NC, NP1, hf), f32),
                     mesh=_MESH, scratch_types=scratch,
                     compiler_params=_SC_PARAMS)


_sc_rows_gcn = {hf: _make_sc_rows_gcn(hf) for hf in (128, 64)}


def _pad2(w, r, c_):
    return jnp.zeros((r, c_), f32).at[:w.shape[0], :w.shape[1]].set(w)


def _pad1(v, n):
    return jnp.zeros((n,), f32).at[:v.shape[0]].set(v)


def kernel(x, edge_index, params):
    src2 = edge_index[0].reshape(E2, K)
    dst2 = edge_index[1].reshape(E2, K)

    g1, g2, g3, gm = (params["gat1"], params["gat2"], params["gat3"],
                      params["gat_mu"])
    w1 = _pad2(g1["W"], 128, D1P)
    w2 = _pad2(g2["W"], D1P, D2P)
    w3 = _pad2(g3["W"], D2P, D3P)
    wm = _pad2(gm["W"], D3P, DMU)
    gat_w = [w1, w2, w3, wm]
    gat_as = [_pad1(p["att_src"], d) for p, d in
              ((g1, D1P), (g2, D2P), (g3, D3P), (gm, DMU))]
    gat_ad = [_pad1(p["att_dst"], d) for p, d in
              ((g1, D1P), (g2, D2P), (g3, D3P), (gm, DMU))]
    gat_b = [_pad1(p["b"], d) for p, d in
             ((g1, D1P), (g2, D2P), (g3, D3P), (gm, DMU))]

    h, asv, adv, ps = tc_gat_pre(x, gat_w[0], gat_as[0], gat_ad[0])
    p, den, deg = _sc_alpha_deg(src2, dst2, asv, adv)
    inv = tc_inv(den, ps)
    for layer in range(4):
        chp = gat_w[layer].shape[1]
        acc = _sc_rows_gat[chp](h, src2, dst2, p, inv)
        if layer < 3:
            h, asv, adv, ps = tc_gat_combine_next(
                acc, h, ps, inv, gat_b[layer],
                gat_w[layer + 1], gat_as[layer + 1], gat_ad[layer + 1])
            p, den = _sc_alpha(src2, dst2, asv, adv)
            inv = tc_inv(den, ps)
        else:
            edges = tc_gat_final(acc, h, ps, inv, gat_b[layer])

    c1, c2, c3, cm = (params["gcn1"], params["gcn2"], params["gcn3"],
                      params["gcn_mu"])
    dinv, ga, gb = tc_gcn_pre(x, c1["W"], deg)
    o = _sc_rows_gcn[128](jnp.stack([ga, gb]), src2, dst2)
    ga, gb = tc_gcn_combine_next(o, ga, gb, dinv, c1["b"], c2["W"])
    o = _sc_rows_gcn[64](jnp.stack([ga, gb]), src2, dst2)
    ga, gb = tc_gcn_combine_next(o, ga, gb, dinv, c2["b"], c3["W"])
    o = _sc_rows_gcn[128](jnp.stack([ga, gb]), src2, dst2)
    ga, gb = tc_gcn_combine_next(o, ga, gb, dinv, c3["b"], cm["W"])
    o = _sc_rows_gcn[64](jnp.stack([ga, gb]), src2, dst2)
    nodes = tc_gcn_final(o, ga, gb, dinv, cm["b"])

    return (edges, nodes)


# speedup vs baseline: 29.7758x; 1.3059x over previous
import functools

import jax
import jax.numpy as jnp
from jax import lax
from jax.experimental import pallas as pl
from jax.experimental.pallas import tpu as pltpu
from jax.experimental.pallas import tpu_sc as plsc

N = 10000
E = 320000
K = 80
E2 = E // K
NC = 2
NS = 16
NW = NC * NS
NP1 = 10240
LANES = 16

D1, D2, D3, DMU = 94, 61, 38, 16
D1P, D2P, D3P = 96, 64, 48

f32 = jnp.float32
i32 = jnp.int32


def _tc(body, out_shape, *args):
    return pl.pallas_call(body, out_shape=out_shape)(*args)


def _gat_head(h, avs, avd):
    asv = jnp.sum(h * avs[None, :], axis=1)
    adv = jnp.sum(h * avd[None, :], axis=1)
    a = asv + adv
    a = jnp.where(a >= 0, a, 0.2 * a)
    return asv, adv, jnp.exp(a)


def tc_gat_pre(x, w, avs, avd):
    chp = w.shape[1]

    def body(x_r, w_r, avs_r, avd_r, h_r, as_r, ad_r, ps_r):
        h = jnp.dot(x_r[...], w_r[...], preferred_element_type=f32)
        h_r[...] = h
        asv, adv, ps = _gat_head(h, avs_r[...], avd_r[...])
        as_r[...] = asv
        ad_r[...] = adv
        ps_r[...] = ps

    return _tc(body, (
        jax.ShapeDtypeStruct((N, chp), f32),
        jax.ShapeDtypeStruct((N,), f32),
        jax.ShapeDtypeStruct((N,), f32),
        jax.ShapeDtypeStruct((N,), f32),
    ), x, w, avs, avd)


def _gat_prev(acc_r, h_r, ps_r, inv_r, b_r):
    wself = ps_r[...] * inv_r[...]
    return (acc_r[0, :N] + acc_r[1, :N]
            + wself[:, None] * h_r[...]) + b_r[...][None, :]


def tc_inv(den, ps):
    def body(den_r, ps_r, inv_r):
        den_t = den_r[0, :N] + den_r[1, :N] + ps_r[...]
        inv_r[...] = 1.0 / jnp.maximum(den_t, 1e-16)

    return _tc(body, jax.ShapeDtypeStruct((N,), f32), den, ps)


def tc_gat_combine_next(acc, h, ps, inv, b, wn, avs, avd):
    chn = wn.shape[1]

    def body(acc_r, h_r, ps_r, inv_r, b_r, wn_r, avs_r, avd_r,
             h2_r, as_r, ad_r, ps2_r):
        prev = _gat_prev(acc_r, h_r, ps_r, inv_r, b_r)
        e = jnp.maximum(prev, 0.0)
        h2 = jnp.dot(e, wn_r[...], preferred_element_type=f32)
        h2_r[...] = h2
        asv, adv, ps2 = _gat_head(h2, avs_r[...], avd_r[...])
        as_r[...] = asv
        ad_r[...] = adv
        ps2_r[...] = ps2

    return _tc(body, (
        jax.ShapeDtypeStruct((N, chn), f32),
        jax.ShapeDtypeStruct((N,), f32),
        jax.ShapeDtypeStruct((N,), f32),
        jax.ShapeDtypeStruct((N,), f32),
    ), acc, h, ps, inv, b, wn, avs, avd)


def tc_gat_final(acc, h, ps, inv, b):
    def body(acc_r, h_r, ps_r, inv_r, b_r, out_r):
        prev = _gat_prev(acc_r, h_r, ps_r, inv_r, b_r)
        out_r[...] = jax.nn.sigmoid(prev)

    return _tc(body, jax.ShapeDtypeStruct((N, DMU), f32), acc, h, ps, inv, b)


def tc_gcn_pre(x, w, deg):
    dout = w.shape[1]
    hf = dout // 2

    def body(x_r, w_r, deg_r, dinv_r, ga_r, gb_r):
        deg_t = deg_r[0, :N] + deg_r[1, :N] + 1.0
        dinv = lax.rsqrt(deg_t)
        dinv_r[...] = dinv
        g = dinv[:, None] * jnp.dot(x_r[...], w_r[...],
                                    preferred_element_type=f32)
        ga_r[...] = g[:, :hf]
        gb_r[...] = g[:, hf:]

    return _tc(body, (
        jax.ShapeDtypeStruct((N,), f32),
        jax.ShapeDtypeStruct((N, hf), f32),
        jax.ShapeDtypeStruct((N, hf), f32),
    ), x, w, deg)


def tc_gcn_combine_next(o, ga, gb, dinv, b, wn):
    dout = wn.shape[1]
    hf = dout // 2

    def body(o_r, ga_r, gb_r, dinv_r, b_r, wn_r, ga2_r, gb2_r):
        sc = jnp.concatenate([o_r[0, :N], o_r[1, :N]], axis=1)
        g = jnp.concatenate([ga_r[...], gb_r[...]], axis=1)
        dinv = dinv_r[...]
        prev = dinv[:, None] * (sc + g) + b_r[...][None, :]
        e = jnp.maximum(prev, 0.0)
        g2 = dinv[:, None] * jnp.dot(e, wn_r[...],
                                     preferred_element_type=f32)
        ga2_r[...] = g2[:, :hf]
        gb2_r[...] = g2[:, hf:]

    return _tc(body, (
        jax.ShapeDtypeStruct((N, hf), f32),
        jax.ShapeDtypeStruct((N, hf), f32),
    ), o, ga, gb, dinv, b, wn)


def tc_gcn_final(o, ga, gb, dinv, b):
    def body(o_r, ga_r, gb_r, dinv_r, b_r, out_r):
        sc = jnp.concatenate([o_r[0, :N], o_r[1, :N]], axis=1)
        g = jnp.concatenate([ga_r[...], gb_r[...]], axis=1)
        prev = dinv_r[...][:, None] * (sc + g) + b_r[...][None, :]
        out_r[...] = jax.nn.sigmoid(prev)

    return _tc(body, jax.ShapeDtypeStruct((N, 2 * ga.shape[1]), f32),
               o, ga, gb, dinv, b)


_MESH = plsc.VectorSubcoreMesh(core_axis_name="c", subcore_axis_name="s")
_SC_PARAMS = pltpu.CompilerParams(use_tc_tiling_on_sc=False,
                                  needs_layout_passes=False)

_RPW_GAT = E2 // NW
_RPW_GCN = E2 // NS
_STRIPE1 = NP1 // NS
_STRIPEN = N // NS


def _zero_1d(zb, acc, s):
    @pl.loop(0, _STRIPE1 // LANES)
    def _(t):
        zb[pl.ds(pl.multiple_of(t * LANES, LANES), LANES)] = jnp.zeros(
            (LANES,), f32)
    pltpu.sync_copy(zb, acc.at[pl.ds(s * _STRIPE1, _STRIPE1)])


def _zero_rows(buf, acc, s, chp):
    @pl.loop(0, K * (chp // LANES))
    def _(t):
        r = t // (chp // LANES)
        v = t % (chp // LANES)
        buf[r, pl.ds(pl.multiple_of(v * LANES, LANES), LANES)] = jnp.zeros(
            (LANES,), f32)

    @pl.loop(0, _STRIPE1 // K)
    def _(t):
        pltpu.sync_copy(buf, acc.at[pl.ds(s * _STRIPE1 + t * K, K)])


def _sc_alpha_body(with_deg, src_h, dst_h, as_h, ad_h, p_h, den_h, deg_h,
                   srcv, dstv, at, dt, pbuf, zb, ones, den_acc, deg_acc):
    c = lax.axis_index("c")
    s = lax.axis_index("s")
    wid = c * NS + s
    row0 = wid * _RPW_GAT
    pltpu.sync_copy(src_h.at[pl.ds(row0, _RPW_GAT)], srcv)
    pltpu.sync_copy(dst_h.at[pl.ds(row0, _RPW_GAT)], dstv)
    pltpu.sync_copy(as_h, at)
    pltpu.sync_copy(ad_h, dt)

    _zero_1d(zb, den_acc, s)
    if with_deg:
        pltpu.sync_copy(zb, deg_acc.at[pl.ds(s * _STRIPE1, _STRIPE1)])
        for g in range(K // LANES):
            ones[pl.ds(g * LANES, LANES)] = jnp.ones((LANES,), f32)
    plsc.subcore_barrier()

    @pl.loop(0, _RPW_GAT)
    def _(i):
        for g in range(K // LANES):
            off = g * LANES
            s16 = srcv[i, pl.ds(off, LANES)]
            d16 = dstv[i, pl.ds(off, LANES)]
            a = plsc.load_gather(at, [s16]) + plsc.load_gather(dt, [d16])
            a = jnp.where(a >= 0, a, 0.2 * a)
            pbuf[i, pl.ds(off, LANES)] = jnp.exp(a)
        pltpu.sync_copy(pbuf.at[i], den_acc.at[dstv.at[i]], add=True)
        if with_deg:
            pltpu.sync_copy(ones, deg_acc.at[dstv.at[i]], add=True)

    plsc.subcore_barrier()
    pltpu.sync_copy(pbuf, p_h.at[pl.ds(row0, _RPW_GAT)])
    pltpu.sync_copy(den_acc.at[pl.ds(s * _STRIPE1, _STRIPE1)],
                    den_h.at[c, pl.ds(s * _STRIPE1, _STRIPE1)])
    if with_deg:
        pltpu.sync_copy(deg_acc.at[pl.ds(s * _STRIPE1, _STRIPE1)],
                        deg_h.at[c, pl.ds(s * _STRIPE1, _STRIPE1)])


def _make_sc_alpha(with_deg):
    outs = [jax.ShapeDtypeStruct((E2, K), f32),
            jax.ShapeDtypeStruct((NC, NP1), f32)]
    if with_deg:
        outs.append(jax.ShapeDtypeStruct((NC, NP1), f32))
    scratch = [
        pltpu.VMEM((_RPW_GAT, K), i32),
        pltpu.VMEM((_RPW_GAT, K), i32),
        pltpu.VMEM((N,), f32),
        pltpu.VMEM((N,), f32),
        pltpu.VMEM((_RPW_GAT, K), f32),
        pltpu.VMEM((_STRIPE1,), f32),
        pltpu.VMEM((K,), f32),
        pltpu.VMEM_SHARED((NP1,), f32),
        pltpu.VMEM_SHARED((NP1,), f32),
    ]

    def body(src_h, dst_h, as_h, ad_h, *rest):
        if with_deg:
            p_h, den_h, deg_h = rest[:3]
            rest = rest[3:]
        else:
            p_h, den_h = rest[:2]
            deg_h = None
            rest = rest[2:]
        _sc_alpha_body(with_deg, src_h, dst_h, as_h, ad_h,
                       p_h, den_h, deg_h, *rest)

    return pl.kernel(body, out_type=tuple(outs), mesh=_MESH,
                     scratch_types=scratch, compiler_params=_SC_PARAMS)


_sc_alpha_deg = _make_sc_alpha(True)
_sc_alpha = _make_sc_alpha(False)


def _make_sc_rows_gat(chp):
    scratch = [
        pltpu.VMEM((_RPW_GAT, K), i32),
        pltpu.VMEM((_RPW_GAT, K), i32),
        pltpu.VMEM((_RPW_GAT, K), f32),
        pltpu.VMEM((N,), f32),
        pltpu.VMEM((K, chp), f32),
        pltpu.VMEM((K, chp), f32),
        pltpu.SemaphoreType.DMA,
        pltpu.SemaphoreType.DMA,
        pltpu.VMEM_SHARED((NP1, chp), f32),
    ]

    def body(h_h, src_h, dst_h, p_h, inv_h, acc_h,
             srcv, dstv, pv, inv, gbuf0, gbuf1, sem0, sem1, acc):
        c = lax.axis_index("c")
        s = lax.axis_index("s")
        wid = c * NS + s
        row0 = wid * _RPW_GAT
        pltpu.sync_copy(src_h.at[pl.ds(row0, _RPW_GAT)], srcv)
        pltpu.sync_copy(dst_h.at[pl.ds(row0, _RPW_GAT)], dstv)
        pltpu.sync_copy(p_h.at[pl.ds(row0, _RPW_GAT)], pv)
        pltpu.sync_copy(inv_h, inv)

        _zero_rows(gbuf0, acc, s, chp)
        plsc.subcore_barrier()

        lane = lax.iota(i32, LANES)

        def scale_and_scatter(i, gbuf):
            for g in range(K // LANES):
                off = g * LANES
                d16 = dstv[i, pl.ds(off, LANES)]
                w16 = pv[i, pl.ds(off, LANES)] * plsc.load_gather(inv, [d16])
                for j in range(LANES):
                    wj = jnp.sum(jnp.where(lane == j, w16, 0.0))
                    e = off + j
                    for v in range(chp // LANES):
                        gbuf[e, pl.ds(v * LANES, LANES)] = (
                            gbuf[e, pl.ds(v * LANES, LANES)] * wj)
            pltpu.sync_copy(gbuf, acc.at[dstv.at[i]], add=True)

        def wait(gbuf, sem):
            pltpu.make_async_copy(h_h.at[pl.ds(0, K)], gbuf, sem).wait()

        pltpu.async_copy(h_h.at[srcv.at[0]], gbuf0, sem0)

        @pl.loop(0, (_RPW_GAT - 1) // 2)
        def _(t):
            i0 = t * 2
            wait(gbuf0, sem0)
            pltpu.async_copy(h_h.at[srcv.at[i0 + 1]], gbuf1, sem1)
            scale_and_scatter(i0, gbuf0)
            wait(gbuf1, sem1)
            pltpu.async_copy(h_h.at[srcv.at[i0 + 2]], gbuf0, sem0)
            scale_and_scatter(i0 + 1, gbuf1)

        wait(gbuf0, sem0)
        scale_and_scatter(_RPW_GAT - 1, gbuf0)

        plsc.subcore_barrier()
        pltpu.sync_copy(acc.at[pl.ds(s * _STRIPE1, _STRIPE1)],
                        acc_h.at[c, pl.ds(s * _STRIPE1, _STRIPE1)])

    return pl.kernel(body,
                     out_type=jax.ShapeDtypeStruct((NC, NP1, chp), f32),
                     mesh=_MESH, scratch_types=scratch,
                     compiler_params=_SC_PARAMS)


_sc_rows_gat = {chp: _make_sc_rows_gat(chp) for chp in (D1P, D2P, D3P, DMU)}


def _make_sc_rows_gcn(hf):
    scratch = [
        pltpu.VMEM((_RPW_GAT, K), i32),
        pltpu.VMEM((_RPW_GAT, K), i32),
        pltpu.VMEM((K, hf), f32),
        pltpu.VMEM((K, hf), f32),
        pltpu.SemaphoreType.DMA,
        pltpu.SemaphoreType.DMA,
        pltpu.VMEM_SHARED((NP1, hf), f32),
    ]

    def body(g_h, src_h, dst_h, o_h, srcv, dstv, gbuf0, gbuf1,
             sem0, sem1, acc):
        c = lax.axis_index("c")
        s = lax.axis_index("s")

        _zero_rows(gbuf0, acc, s, hf)
        plsc.subcore_barrier()

        def wait(gbuf, sem):
            pltpu.make_async_copy(g_h.at[0].at[pl.ds(0, K)], gbuf, sem).wait()

        @pl.loop(0, _RPW_GCN // _RPW_GAT)
        def _(ph):
            row0 = s * _RPW_GCN + ph * _RPW_GAT
            pltpu.sync_copy(src_h.at[pl.ds(row0, _RPW_GAT)], srcv)
            pltpu.sync_copy(dst_h.at[pl.ds(row0, _RPW_GAT)], dstv)

            pltpu.async_copy(g_h.at[c].at[srcv.at[0]], gbuf0, sem0)

            @pl.loop(0, (_RPW_GAT - 1) // 2)
            def _(t):
                i0 = t * 2
                wait(gbuf0, sem0)
                pltpu.async_copy(g_h.at[c].at[srcv.at[i0 + 1]], gbuf1, sem1)
                pltpu.sync_copy(gbuf0, acc.at[dstv.at[i0]], add=True)
                wait(gbuf1, sem1)
                pltpu.async_copy(g_h.at[c].at[srcv.at[i0 + 2]], gbuf0, sem0)
                pltpu.sync_copy(gbuf1, acc.at[dstv.at[i0 + 1]], add=True)

            wait(gbuf0, sem0)
            pltpu.sync_copy(gbuf0, acc.at[dstv.at[_RPW_GAT - 1]], add=True)

        plsc.subcore_barrier()
        pltpu.sync_copy(acc.at[pl.ds(s * _STRIPE1, _STRIPE1)],
                        o_h.at[c, pl.ds(s * _STRIPE1, _STRIPE1)])

    return pl.kernel(body,
                     out_type=jax.ShapeDtypeStruct((NC, NP1, hf), f32),
                     mesh=_MESH, scratch_types=scratch,
                     compiler_params=_SC_PARAMS)


_sc_rows_gcn = {hf: _make_sc_rows_gcn(hf) for hf in (128, 64)}


def _pad2(w, r, c_):
    return jnp.zeros((r, c_), f32).at[:w.shape[0], :w.shape[1]].set(w)


def _pad1(v, n):
    return jnp.zeros((n,), f32).at[:v.shape[0]].set(v)


def kernel(x, edge_index, params):
    src2 = edge_index[0].reshape(E2, K)
    dst2 = edge_index[1].reshape(E2, K)

    g1, g2, g3, gm = (params["gat1"], params["gat2"], params["gat3"],
                      params["gat_mu"])
    w1 = _pad2(g1["W"], 128, D1P)
    w2 = _pad2(g2["W"], D1P, D2P)
    w3 = _pad2(g3["W"], D2P, D3P)
    wm = _pad2(gm["W"], D3P, DMU)
    gat_w = [w1, w2, w3, wm]
    gat_as = [_pad1(p["att_src"], d) for p, d in
              ((g1, D1P), (g2, D2P), (g3, D3P), (gm, DMU))]
    gat_ad = [_pad1(p["att_dst"], d) for p, d in
              ((g1, D1P), (g2, D2P), (g3, D3P), (gm, DMU))]
    gat_b = [_pad1(p["b"], d) for p, d in
             ((g1, D1P), (g2, D2P), (g3, D3P), (gm, DMU))]

    h, asv, adv, ps = tc_gat_pre(x, gat_w[0], gat_as[0], gat_ad[0])
    p, den, deg = _sc_alpha_deg(src2, dst2, asv, adv)
    inv = tc_inv(den, ps)
    for layer in range(4):
        chp = gat_w[layer].shape[1]
        acc = _sc_rows_gat[chp](h, src2, dst2, p, inv)
        if layer < 3:
            h, asv, adv, ps = tc_gat_combine_next(
                acc, h, ps, inv, gat_b[layer],
                gat_w[layer + 1], gat_as[layer + 1], gat_ad[layer + 1])
            p, den = _sc_alpha(src2, dst2, asv, adv)
            inv = tc_inv(den, ps)
        else:
            edges = tc_gat_final(acc, h, ps, inv, gat_b[layer])

    c1, c2, c3, cm = (params["gcn1"], params["gcn2"], params["gcn3"],
                      params["gcn_mu"])
    dinv, ga, gb = tc_gcn_pre(x, c1["W"], deg)
    o = _sc_rows_gcn[128](jnp.stack([ga, gb]), src2, dst2)
    ga, gb = tc_gcn_combine_next(o, ga, gb, dinv, c1["b"], c2["W"])
    o = _sc_rows_gcn[64](jnp.stack([ga, gb]), src2, dst2)
    ga, gb = tc_gcn_combine_next(o, ga, gb, dinv, c2["b"], c3["W"])
    o = _sc_rows_gcn[128](jnp.stack([ga, gb]), src2, dst2)
    ga, gb = tc_gcn_combine_next(o, ga, gb, dinv, c3["b"], cm["W"])
    o = _sc_rows_gcn[64](jnp.stack([ga, gb]), src2, dst2)
    nodes = tc_gcn_final(o, ga, gb, dinv, cm["b"])

    return (edges, nodes)
